# baseline (speedup 1.0000x reference)
import functools

import jax
import jax.numpy as jnp
import numpy as np
from jax import lax
from jax.experimental import pallas as pl
from jax.experimental.pallas import tpu as pltpu
from jax.experimental.pallas import tpu_sc as plsc

_NUM_FIELDS = 26
_FIELD_DIM = 40000
_BATCH = 16384
_EMBED_DIM = 16
_TOTAL = _BATCH * _NUM_FIELDS
_NUM_WORKERS = 32
_PER_WORKER = _TOTAL // _NUM_WORKERS
_CHUNK = 1024
_NUM_CHUNKS = _PER_WORKER // _CHUNK
_IDX_ROWS = _CHUNK // 128
_PAT = 208

_OFFS_PATTERN = np.tile(
    (np.arange(_NUM_FIELDS, dtype=np.int32) * _FIELD_DIM), _PAT // _NUM_FIELDS
)


def _sc_embedding_lookup(x2d, offs_pat, table):
    mesh = plsc.VectorSubcoreMesh(core_axis_name="c", subcore_axis_name="s")

    @functools.partial(
        pl.kernel,
        mesh=mesh,
        compiler_params=pltpu.CompilerParams(use_tc_tiling_on_sc=False),
        out_type=jax.ShapeDtypeStruct((_TOTAL, _EMBED_DIM), jnp.float32),
        scratch_types=[
            pltpu.VMEM((_PAT,), jnp.int32),
            pltpu.VMEM((_IDX_ROWS, 128), jnp.int32),
            pltpu.VMEM((_CHUNK, _EMBED_DIM), jnp.float32),
            pltpu.SemaphoreType.DMA,
        ],
    )
    def k(x_hbm, offs_hbm, table_hbm, out_hbm, offs_v, idx_v, rows_v, sem):
        wid = lax.axis_index("s") * 2 + lax.axis_index("c")
        pltpu.sync_copy(offs_hbm, offs_v)
        for c in range(_NUM_CHUNKS):
            base = wid * _PER_WORKER + c * _CHUNK
            row0 = wid * (_PER_WORKER // 128) + c * _IDX_ROWS
            pltpu.sync_copy(x_hbm.at[pl.ds(row0, _IDX_ROWS)], idx_v)
            for j in range(_IDX_ROWS):
                for i in range(8):
                    pat = ((64 * c + 8 * j + i) % 13) * 16
                    idx_v[j, pl.ds(i * 16, 16)] = (
                        idx_v[j, pl.ds(i * 16, 16)] + offs_v[pl.ds(pat, 16)]
                    )
            copies = [
                pltpu.async_copy(
                    table_hbm.at[idx_v.at[j]],
                    rows_v.at[pl.ds(j * 128, 128)],
                    sem,
                )
                for j in range(_IDX_ROWS)
            ]
            for cp in copies:
                cp.wait()
            pltpu.sync_copy(rows_v, out_hbm.at[pl.ds(base, _CHUNK)])

    return k(x2d, offs_pat, table)


def kernel(x, table):
    x2d = x.astype(jnp.int32).reshape(_TOTAL // 128, 128)
    offs_pat = jnp.asarray(_OFFS_PATTERN)
    out = _sc_embedding_lookup(x2d, offs_pat, table)
    return out.reshape(_BATCH, _NUM_FIELDS, _EMBED_DIM)


# speedup vs baseline: 1.0114x; 1.0114x over previous
import functools

import jax
import jax.numpy as jnp
import numpy as np
from jax import lax
from jax.experimental import pallas as pl
from jax.experimental.pallas import tpu as pltpu
from jax.experimental.pallas import tpu_sc as plsc

_NUM_FIELDS = 26
_FIELD_DIM = 40000
_BATCH = 16384
_EMBED_DIM = 16
_TOTAL = _BATCH * _NUM_FIELDS
_NUM_WORKERS = 32
_PER_WORKER = _TOTAL // _NUM_WORKERS
_CHUNK = 1024
_NUM_CHUNKS = _PER_WORKER // _CHUNK
_IDX_ROWS = _CHUNK // 128
_PAT = 208

_OFFS_PATTERN = np.tile(
    (np.arange(_NUM_FIELDS, dtype=np.int32) * _FIELD_DIM), _PAT // _NUM_FIELDS
)


def _sc_embedding_lookup(x2d, offs_pat, table):
    mesh = plsc.VectorSubcoreMesh(core_axis_name="c", subcore_axis_name="s")

    @functools.partial(
        pl.kernel,
        mesh=mesh,
        compiler_params=pltpu.CompilerParams(use_tc_tiling_on_sc=False),
        out_type=jax.ShapeDtypeStruct((_TOTAL, _EMBED_DIM), jnp.float32),
        scratch_types=[
            pltpu.VMEM((_PAT,), jnp.int32),
            pltpu.VMEM((2, _IDX_ROWS, 128), jnp.int32),
            pltpu.VMEM((2, _CHUNK, _EMBED_DIM), jnp.float32),
            pltpu.SemaphoreType.DMA,
            pltpu.SemaphoreType.DMA,
            pltpu.SemaphoreType.DMA,
        ],
    )
    def k(x_hbm, offs_hbm, table_hbm, out_hbm,
          offs_v, idx_v, rows_v, idx_sem, gat_sem, out_sem):
        wid = lax.axis_index("s") * 2 + lax.axis_index("c")
        pltpu.sync_copy(offs_hbm, offs_v)
        row_base = wid * (_PER_WORKER // 128)

        def idx_copy(c, buf):
            return pltpu.async_copy(
                x_hbm.at[pl.ds(row_base + c * _IDX_ROWS, _IDX_ROWS)],
                idx_v.at[buf], idx_sem)

        out_copies = []
        pending_idx = idx_copy(0, 0)
        for c in range(_NUM_CHUNKS):
            cur, nxt = c % 2, (c + 1) % 2
            pending_idx.wait()
            if c + 1 < _NUM_CHUNKS:
                pending_idx = idx_copy(c + 1, nxt)
            for j in range(_IDX_ROWS):
                for i in range(8):
                    pat = ((64 * c + 8 * j + i) % 13) * 16
                    idx_v[cur, j, pl.ds(i * 16, 16)] = (
                        idx_v[cur, j, pl.ds(i * 16, 16)]
                        + offs_v[pl.ds(pat, 16)]
                    )
            if c >= 2:
                out_copies[c - 2].wait()
            gathers = [
                pltpu.async_copy(
                    table_hbm.at[idx_v.at[cur, j]],
                    rows_v.at[cur, pl.ds(j * 128, 128)],
                    gat_sem,
                )
                for j in range(_IDX_ROWS)
            ]
            for cp in gathers:
                cp.wait()
            out_copies.append(pltpu.async_copy(
                rows_v.at[cur],
                out_hbm.at[pl.ds(wid * _PER_WORKER + c * _CHUNK, _CHUNK)],
                out_sem))
        out_copies[_NUM_CHUNKS - 2].wait()
        out_copies[_NUM_CHUNKS - 1].wait()

    return k(x2d, offs_pat, table)


def kernel(x, table):
    x2d = x.astype(jnp.int32).reshape(_TOTAL // 128, 128)
    offs_pat = jnp.asarray(_OFFS_PATTERN)
    out = _sc_embedding_lookup(x2d, offs_pat, table)
    return out.reshape(_BATCH, _NUM_FIELDS, _EMBED_DIM)


# speedup vs baseline: 1.0117x; 1.0003x over previous
import functools

import jax
import jax.numpy as jnp
import numpy as np
from jax import lax
from jax.experimental import pallas as pl
from jax.experimental.pallas import tpu as pltpu
from jax.experimental.pallas import tpu_sc as plsc

_NUM_FIELDS = 26
_FIELD_DIM = 40000
_BATCH = 16384
_EMBED_DIM = 16
_TOTAL = _BATCH * _NUM_FIELDS
_NUM_WORKERS = 32
_PER_WORKER = _TOTAL // _NUM_WORKERS
_CHUNK = 1024
_NUM_CHUNKS = _PER_WORKER // _CHUNK
_IDX_ROWS = _CHUNK // 128
_PAT = 208

_OFFS_PATTERN = np.tile(
    (np.arange(_NUM_FIELDS, dtype=np.int32) * _FIELD_DIM), _PAT // _NUM_FIELDS
)


def _sc_embedding_lookup(x2d, offs_pat, table):
    mesh = plsc.VectorSubcoreMesh(core_axis_name="c", subcore_axis_name="s")

    @functools.partial(
        pl.kernel,
        mesh=mesh,
        compiler_params=pltpu.CompilerParams(
            use_tc_tiling_on_sc=False, skip_device_barrier=True),
        out_type=jax.ShapeDtypeStruct((_TOTAL, _EMBED_DIM), jnp.float32),
        scratch_types=[
            pltpu.VMEM((_PAT,), jnp.int32),
            pltpu.VMEM((2, _IDX_ROWS, 128), jnp.int32),
            pltpu.VMEM((2, _CHUNK, _EMBED_DIM), jnp.float32),
            pltpu.SemaphoreType.DMA,
            pltpu.SemaphoreType.DMA,
            pltpu.SemaphoreType.DMA,
        ],
    )
    def k(x_hbm, offs_hbm, table_hbm, out_hbm,
          offs_v, idx_v, rows_v, idx_sem, gat_sem, out_sem):
        wid = lax.axis_index("s") * 2 + lax.axis_index("c")
        pltpu.sync_copy(offs_hbm, offs_v)
        row_base = wid * (_PER_WORKER // 128)

        def idx_copy(c, buf):
            return pltpu.async_copy(
                x_hbm.at[pl.ds(row_base + c * _IDX_ROWS, _IDX_ROWS)],
                idx_v.at[buf], idx_sem)

        out_copies = []
        pending_idx = idx_copy(0, 0)
        for c in range(_NUM_CHUNKS):
            cur, nxt = c % 2, (c + 1) % 2
            pending_idx.wait()
            if c + 1 < _NUM_CHUNKS:
                pending_idx = idx_copy(c + 1, nxt)
            for j in range(_IDX_ROWS):
                for i in range(8):
                    pat = ((64 * c + 8 * j + i) % 13) * 16
                    idx_v[cur, j, pl.ds(i * 16, 16)] = (
                        idx_v[cur, j, pl.ds(i * 16, 16)]
                        + offs_v[pl.ds(pat, 16)]
                    )
            if c >= 2:
                out_copies[c - 2].wait()
            gathers = [
                pltpu.async_copy(
                    table_hbm.at[idx_v.at[cur, j]],
                    rows_v.at[cur, pl.ds(j * 128, 128)],
                    gat_sem,
                )
                for j in range(_IDX_ROWS)
            ]
            for cp in gathers:
                cp.wait()
            out_copies.append(pltpu.async_copy(
                rows_v.at[cur],
                out_hbm.at[pl.ds(wid * _PER_WORKER + c * _CHUNK, _CHUNK)],
                out_sem))
        out_copies[_NUM_CHUNKS - 2].wait()
        out_copies[_NUM_CHUNKS - 1].wait()

    return k(x2d, offs_pat, table)


def kernel(x, table):
    x2d = x.astype(jnp.int32).reshape(_TOTAL // 128, 128)
    offs_pat = jnp.asarray(_OFFS_PATTERN)
    out = _sc_embedding_lookup(x2d, offs_pat, table)
    return out.reshape(_BATCH, _NUM_FIELDS, _EMBED_DIM)
